# Initial kernel scaffold; baseline (speedup 1.0000x reference)
import jax
import jax.numpy as jnp
from jax.experimental import pallas as pl


def kernel(hidden_states, w_router, w_up, w_down):
    raise NotImplementedError("write your pallas kernel here")


# speedup vs baseline: 1.6800x; 1.6800x over previous
import functools

import jax
import jax.numpy as jnp
from jax import lax
from jax.experimental import pallas as pl
from jax.experimental.pallas import tpu as pltpu
from jax.experimental.pallas import tpu_sc as plsc

E = 8
TOPK = 2
D_MODEL = 1024
D_FF = 2048
T = 2048
CAP = (T * TOPK // E) * 2
NROWS = E * CAP
TRASH = NROWS
DISP_ROWS = NROWS + 256

NW = 32
TPW = T // NW
CHUNK = 32
BC = 256
NC_BLK = CAP // BC


def _router_body(h_ref, wr_ref, s1_ref, s2_ref, r1_ref, r2_ref,
                 w1_ref, w2_ref, cnt_ref):
    logits = jnp.dot(h_ref[...], wr_ref[...], preferred_element_type=jnp.float32)
    probs = jax.nn.softmax(logits, axis=-1)
    col = lax.broadcasted_iota(jnp.int32, (T, E), 1)
    m1 = jnp.max(probs, axis=1, keepdims=True)
    i1 = jnp.min(jnp.where(probs == m1, col, E), axis=1, keepdims=True)
    masked = jnp.where(col == i1, -1.0, probs)
    m2 = jnp.max(masked, axis=1, keepdims=True)
    i2 = jnp.min(jnp.where(masked == m2, col, E), axis=1, keepdims=True)

    o1 = (col == i1).astype(jnp.float32)
    o2 = (col == i2).astype(jnp.float32)
    occ = o1 + o2
    incl = occ
    sh = 1
    while sh < T:
        shifted = jnp.concatenate(
            [jnp.zeros((sh, E), jnp.float32), incl[:-sh, :]], axis=0)
        incl = incl + shifted
        sh *= 2
    excl = incl - occ
    rank1 = jnp.sum(excl * o1, axis=1, keepdims=True).astype(jnp.int32)
    rank2 = jnp.sum(excl * o2, axis=1, keepdims=True).astype(jnp.int32)

    keep1 = rank1 < CAP
    keep2 = rank2 < CAP
    s1_ref[...] = jnp.where(keep1, i1 * CAP + rank1, TRASH)
    s2_ref[...] = jnp.where(keep2, i2 * CAP + rank2, TRASH)
    r1_ref[...] = i1 * CAP + jnp.minimum(rank1, CAP - 1)
    r2_ref[...] = i2 * CAP + jnp.minimum(rank2, CAP - 1)
    w1_ref[...] = jnp.broadcast_to(m1 * keep1.astype(jnp.float32), (T, 16))
    w2_ref[...] = jnp.broadcast_to(m2 * keep2.astype(jnp.float32), (T, 16))
    cnt_ref[...] = incl[T - 1:T, :].astype(jnp.int32)


def _router(hidden, w_router):
    return pl.pallas_call(
        _router_body,
        out_shape=(
            jax.ShapeDtypeStruct((T, 1), jnp.int32),
            jax.ShapeDtypeStruct((T, 1), jnp.int32),
            jax.ShapeDtypeStruct((T, 1), jnp.int32),
            jax.ShapeDtypeStruct((T, 1), jnp.int32),
            jax.ShapeDtypeStruct((T, 16), jnp.float32),
            jax.ShapeDtypeStruct((T, 16), jnp.float32),
            jax.ShapeDtypeStruct((1, E), jnp.int32),
        ),
    )(hidden, w_router)


@functools.lru_cache(maxsize=None)
def _sc_mesh():
    return plsc.VectorSubcoreMesh(
        core_axis_name="c", subcore_axis_name="s", num_cores=2,
        num_subcores=16)


@functools.lru_cache(maxsize=None)
def _dispatch_kernel():
    @functools.partial(
        pl.kernel,
        out_type=jax.ShapeDtypeStruct((DISP_ROWS, D_MODEL), jnp.float32),
        mesh=_sc_mesh(),
        scratch_types=[
            pltpu.VMEM((TPW,), jnp.int32),
            pltpu.VMEM((TPW,), jnp.int32),
            pltpu.VMEM((TPW, D_MODEL), jnp.float32),
            pltpu.SemaphoreType.DMA,
            pltpu.SemaphoreType.DMA,
        ],
    )
    def _dispatch(h_hbm, s1_hbm, s2_hbm, disp_hbm, idx1_v, idx2_v, rows_v,
                  sem1, sem2):
        wid = lax.axis_index("s") * 2 + lax.axis_index("c")
        base = wid * TPW
        pltpu.sync_copy(s1_hbm.at[pl.ds(base, TPW)], idx1_v)
        pltpu.sync_copy(s2_hbm.at[pl.ds(base, TPW)], idx2_v)
        pltpu.sync_copy(h_hbm.at[pl.ds(base, TPW)], rows_v)
        c1 = pltpu.async_copy(rows_v, disp_hbm.at[idx1_v], sem1)
        c2 = pltpu.async_copy(rows_v, disp_hbm.at[idx2_v], sem2)
        c1.wait()
        c2.wait()

    return _dispatch


def _gemm_body(cnt_ref, x_ref, wu_ref, wd_ref, y_ref):
    e = pl.program_id(0)
    c = pl.program_id(1)
    cnt = cnt_ref[e]
    start = c * BC

    @pl.when(start >= cnt)
    def _():
        y_ref[...] = jnp.zeros_like(y_ref)

    @pl.when(start < cnt)
    def _():
        rows = lax.broadcasted_iota(jnp.int32, (BC, 1), 0) + start
        x = jnp.where(rows < cnt, x_ref[...], 0.0)
        acc = jnp.zeros((BC, D_MODEL), jnp.float32)
        for f in range(4):
            wu = wu_ref[0, :, f * 512:(f + 1) * 512]
            wd = wd_ref[0, f * 512:(f + 1) * 512, :]
            h = jax.nn.gelu(jnp.dot(x, wu, preferred_element_type=jnp.float32))
            acc = acc + jnp.dot(h, wd, preferred_element_type=jnp.float32)
        y_ref[...] = acc


def _gemm(counts, disp, w_up, w_down):
    return pl.pallas_call(
        _gemm_body,
        grid=(E, NC_BLK),
        in_specs=[
            pl.BlockSpec(memory_space=pltpu.SMEM),
            pl.BlockSpec((BC, D_MODEL), lambda e, c: (e * NC_BLK + c, 0)),
            pl.BlockSpec((1, D_MODEL, D_FF), lambda e, c: (e, 0, 0)),
            pl.BlockSpec((1, D_FF, D_MODEL), lambda e, c: (e, 0, 0)),
        ],
        out_specs=pl.BlockSpec((BC, D_MODEL), lambda e, c: (e * NC_BLK + c, 0)),
        out_shape=jax.ShapeDtypeStruct((NROWS, D_MODEL), jnp.float32),
    )(counts, disp, w_up, w_down)


@functools.lru_cache(maxsize=None)
def _combine_kernel():
    @functools.partial(
        pl.kernel,
        out_type=jax.ShapeDtypeStruct((T, D_MODEL), jnp.float32),
        mesh=_sc_mesh(),
        scratch_types=[
            pltpu.VMEM((CHUNK,), jnp.int32),
            pltpu.VMEM((CHUNK,), jnp.int32),
            pltpu.VMEM((TPW, 16), jnp.float32),
            pltpu.VMEM((TPW, 16), jnp.float32),
            pltpu.VMEM((CHUNK, D_MODEL), jnp.float32),
            pltpu.VMEM((CHUNK, D_MODEL), jnp.float32),
            pltpu.VMEM((CHUNK, D_MODEL), jnp.float32),
            pltpu.SemaphoreType.DMA,
            pltpu.SemaphoreType.DMA,
        ],
    )
    def _combine(ys_hbm, r1_hbm, r2_hbm, w1_hbm, w2_hbm, out_hbm,
                 idx1_v, idx2_v, w1_v, w2_v, bufa, bufb, outb, sema, semb):
        wid = lax.axis_index("s") * 2 + lax.axis_index("c")
        base = wid * TPW
        pltpu.sync_copy(w1_hbm.at[pl.ds(base, TPW)], w1_v)
        pltpu.sync_copy(w2_hbm.at[pl.ds(base, TPW)], w2_v)
        for ch in range(TPW // CHUNK):
            cb = base + ch * CHUNK
            pltpu.sync_copy(r1_hbm.at[pl.ds(cb, CHUNK)], idx1_v)
            pltpu.sync_copy(r2_hbm.at[pl.ds(cb, CHUNK)], idx2_v)
            ca = pltpu.async_copy(ys_hbm.at[idx1_v], bufa, sema)
            cbb = pltpu.async_copy(ys_hbm.at[idx2_v], bufb, semb)
            ca.wait()
            cbb.wait()

            def row(j, _):
                wa = w1_v[ch * CHUNK + j, :]
                wb = w2_v[ch * CHUNK + j, :]

                def lane(cc, __):
                    sl = pl.ds(cc * 16, 16)
                    outb[j, sl] = bufa[j, sl] * wa + bufb[j, sl] * wb
                    return 0

                return lax.fori_loop(0, D_MODEL // 16, lane, 0)

            lax.fori_loop(0, CHUNK, row, 0)
            pltpu.sync_copy(outb, out_hbm.at[pl.ds(cb, CHUNK)])

    return _combine


@jax.jit
def kernel(hidden_states, w_router, w_up, w_down):
    s1, s2, r1, r2, w1, w2, counts = _router(hidden_states, w_router)
    s1 = s1.reshape(T)
    s2 = s2.reshape(T)
    r1 = r1.reshape(T)
    r2 = r2.reshape(T)
    counts = counts.reshape(E)
    disp = _dispatch_kernel()(hidden_states, s1, s2)
    ys = _gemm(counts, disp, w_up, w_down)
    return _combine_kernel()(ys, r1, r2, w1, w2)


# speedup vs baseline: 1.7357x; 1.0331x over previous
import functools

import jax
import jax.numpy as jnp
from jax import lax
from jax.experimental import pallas as pl
from jax.experimental.pallas import tpu as pltpu
from jax.experimental.pallas import tpu_sc as plsc

E = 8
TOPK = 2
D_MODEL = 1024
D_FF = 2048
T = 2048
CAP = (T * TOPK // E) * 2
NROWS = E * CAP
TRASH = NROWS
DISP_ROWS = NROWS + 256

NW = 32
TPW = T // NW
CHUNK = 32
BC = 256
NC_BLK = CAP // BC


def _router_body(h_ref, wr_ref, s1_ref, s2_ref, r1_ref, r2_ref,
                 w1_ref, w2_ref, cnt_ref):
    logits = jnp.dot(h_ref[...], wr_ref[...], preferred_element_type=jnp.float32)
    probs = jax.nn.softmax(logits, axis=-1)
    col = lax.broadcasted_iota(jnp.int32, (T, E), 1)
    m1 = jnp.max(probs, axis=1, keepdims=True)
    i1 = jnp.min(jnp.where(probs == m1, col, E), axis=1, keepdims=True)
    masked = jnp.where(col == i1, -1.0, probs)
    m2 = jnp.max(masked, axis=1, keepdims=True)
    i2 = jnp.min(jnp.where(masked == m2, col, E), axis=1, keepdims=True)

    o1 = (col == i1).astype(jnp.float32)
    o2 = (col == i2).astype(jnp.float32)
    occ = o1 + o2
    incl = occ
    sh = 1
    while sh < T:
        shifted = jnp.concatenate(
            [jnp.zeros((sh, E), jnp.float32), incl[:-sh, :]], axis=0)
        incl = incl + shifted
        sh *= 2
    excl = incl - occ
    rank1 = jnp.sum(excl * o1, axis=1, keepdims=True).astype(jnp.int32)
    rank2 = jnp.sum(excl * o2, axis=1, keepdims=True).astype(jnp.int32)

    keep1 = rank1 < CAP
    keep2 = rank2 < CAP
    s1_ref[...] = jnp.where(keep1, i1 * CAP + rank1, TRASH)
    s2_ref[...] = jnp.where(keep2, i2 * CAP + rank2, TRASH)
    r1_ref[...] = i1 * CAP + jnp.minimum(rank1, CAP - 1)
    r2_ref[...] = i2 * CAP + jnp.minimum(rank2, CAP - 1)
    w1_ref[...] = jnp.broadcast_to(m1 * keep1.astype(jnp.float32), (T, 16))
    w2_ref[...] = jnp.broadcast_to(m2 * keep2.astype(jnp.float32), (T, 16))
    cnt_ref[...] = incl[T - 1:T, :].astype(jnp.int32)


def _router(hidden, w_router):
    return pl.pallas_call(
        _router_body,
        out_shape=(
            jax.ShapeDtypeStruct((T, 1), jnp.int32),
            jax.ShapeDtypeStruct((T, 1), jnp.int32),
            jax.ShapeDtypeStruct((T, 1), jnp.int32),
            jax.ShapeDtypeStruct((T, 1), jnp.int32),
            jax.ShapeDtypeStruct((T, 16), jnp.float32),
            jax.ShapeDtypeStruct((T, 16), jnp.float32),
            jax.ShapeDtypeStruct((1, E), jnp.int32),
        ),
    )(hidden, w_router)


@functools.lru_cache(maxsize=None)
def _sc_mesh():
    return plsc.VectorSubcoreMesh(
        core_axis_name="c", subcore_axis_name="s", num_cores=2,
        num_subcores=16)


@functools.lru_cache(maxsize=None)
def _dispatch_kernel():
    @functools.partial(
        pl.kernel,
        out_type=jax.ShapeDtypeStruct((DISP_ROWS, D_MODEL), jnp.float32),
        mesh=_sc_mesh(),
        scratch_types=[
            pltpu.VMEM((TPW,), jnp.int32),
            pltpu.VMEM((TPW,), jnp.int32),
            pltpu.VMEM((TPW, D_MODEL), jnp.float32),
            pltpu.SemaphoreType.DMA,
            pltpu.SemaphoreType.DMA,
        ],
    )
    def _dispatch(h_hbm, s1_hbm, s2_hbm, disp_hbm, idx1_v, idx2_v, rows_v,
                  sem1, sem2):
        wid = lax.axis_index("s") * 2 + lax.axis_index("c")
        base = wid * TPW
        pltpu.sync_copy(s1_hbm.at[pl.ds(base, TPW)], idx1_v)
        pltpu.sync_copy(s2_hbm.at[pl.ds(base, TPW)], idx2_v)
        pltpu.sync_copy(h_hbm.at[pl.ds(base, TPW)], rows_v)
        c1 = pltpu.async_copy(rows_v, disp_hbm.at[idx1_v], sem1)
        c2 = pltpu.async_copy(rows_v, disp_hbm.at[idx2_v], sem2)
        c1.wait()
        c2.wait()

    return _dispatch


def _gemm_body(cnt_ref, x_ref, wu_ref, wd_ref, y_ref):
    e = pl.program_id(0)
    c = pl.program_id(1)
    cnt = cnt_ref[e]
    start = c * BC

    @pl.when(start >= cnt)
    def _():
        y_ref[...] = jnp.zeros_like(y_ref)

    @pl.when(start < cnt)
    def _():
        rows = lax.broadcasted_iota(jnp.int32, (BC, 1), 0) + start
        x = jnp.where(rows < cnt, x_ref[...], 0.0)
        acc = jnp.zeros((BC, D_MODEL), jnp.float32)
        for f in range(4):
            wu = wu_ref[0, :, f * 512:(f + 1) * 512]
            wd = wd_ref[0, f * 512:(f + 1) * 512, :]
            h = jax.nn.gelu(jnp.dot(x, wu, preferred_element_type=jnp.float32))
            acc = acc + jnp.dot(h, wd, preferred_element_type=jnp.float32)
        y_ref[...] = acc


def _gemm(counts, disp, w_up, w_down):
    return pl.pallas_call(
        _gemm_body,
        grid=(E, NC_BLK),
        in_specs=[
            pl.BlockSpec(memory_space=pltpu.SMEM),
            pl.BlockSpec((BC, D_MODEL), lambda e, c: (e * NC_BLK + c, 0)),
            pl.BlockSpec((1, D_MODEL, D_FF), lambda e, c: (e, 0, 0)),
            pl.BlockSpec((1, D_FF, D_MODEL), lambda e, c: (e, 0, 0)),
        ],
        out_specs=pl.BlockSpec((BC, D_MODEL), lambda e, c: (e * NC_BLK + c, 0)),
        out_shape=jax.ShapeDtypeStruct((NROWS, D_MODEL), jnp.float32),
    )(counts, disp, w_up, w_down)


@functools.lru_cache(maxsize=None)
def _combine_kernel():
    @functools.partial(
        pl.kernel,
        out_type=jax.ShapeDtypeStruct((T, D_MODEL), jnp.float32),
        mesh=_sc_mesh(),
        scratch_types=[
            pltpu.VMEM((CHUNK,), jnp.int32),
            pltpu.VMEM((CHUNK,), jnp.int32),
            pltpu.VMEM((TPW, 16), jnp.float32),
            pltpu.VMEM((TPW, 16), jnp.float32),
            pltpu.VMEM((CHUNK, D_MODEL), jnp.float32),
            pltpu.VMEM((CHUNK, D_MODEL), jnp.float32),
            pltpu.VMEM((CHUNK, D_MODEL), jnp.float32),
            pltpu.SemaphoreType.DMA,
            pltpu.SemaphoreType.DMA,
        ],
    )
    def _combine(ys_hbm, r1_hbm, r2_hbm, w1_hbm, w2_hbm, out_hbm,
                 idx1_v, idx2_v, w1_v, w2_v, bufa, bufb, outb, sema, semb):
        wid = lax.axis_index("s") * 2 + lax.axis_index("c")
        base = wid * TPW
        pltpu.sync_copy(w1_hbm.at[pl.ds(base, TPW)], w1_v)
        pltpu.sync_copy(w2_hbm.at[pl.ds(base, TPW)], w2_v)
        for ch in range(TPW // CHUNK):
            cb = base + ch * CHUNK
            pltpu.sync_copy(r1_hbm.at[pl.ds(cb, CHUNK)], idx1_v)
            pltpu.sync_copy(r2_hbm.at[pl.ds(cb, CHUNK)], idx2_v)
            ca = pltpu.async_copy(ys_hbm.at[idx1_v], bufa, sema)
            cbb = pltpu.async_copy(ys_hbm.at[idx2_v], bufb, semb)
            ca.wait()
            cbb.wait()

            def row(j, _):
                wa = w1_v[ch * CHUNK + j, :]
                wb = w2_v[ch * CHUNK + j, :]
                for cc in range(D_MODEL // 16):
                    sl = pl.ds(cc * 16, 16)
                    outb[j, sl] = bufa[j, sl] * wa + bufb[j, sl] * wb
                return 0

            lax.fori_loop(0, CHUNK, row, 0)
            pltpu.sync_copy(outb, out_hbm.at[pl.ds(cb, CHUNK)])

    return _combine


@jax.jit
def kernel(hidden_states, w_router, w_up, w_down):
    s1, s2, r1, r2, w1, w2, counts = _router(hidden_states, w_router)
    s1 = s1.reshape(T)
    s2 = s2.reshape(T)
    r1 = r1.reshape(T)
    r2 = r2.reshape(T)
    counts = counts.reshape(E)
    disp = _dispatch_kernel()(hidden_states, s1, s2)
    ys = _gemm(counts, disp, w_up, w_down)
    return _combine_kernel()(ys, r1, r2, w1, w2)


# speedup vs baseline: 2.2563x; 1.2999x over previous
import functools

import jax
import jax.numpy as jnp
from jax import lax
from jax.experimental import pallas as pl
from jax.experimental.pallas import tpu as pltpu
from jax.experimental.pallas import tpu_sc as plsc

E = 8
TOPK = 2
D_MODEL = 1024
D_FF = 2048
T = 2048
CAP = (T * TOPK // E) * 2
NROWS = E * CAP
TRASH = NROWS
DISP_ROWS = NROWS + 256

NW = 32
TPW = T // NW
CHUNK = 32
BC = 256
NC_BLK = CAP // BC


def _router_body(h_ref, wr_ref, s1_ref, s2_ref, r1_ref, r2_ref,
                 w1_ref, w2_ref, cnt_ref):
    logits = jnp.dot(h_ref[...], wr_ref[...], preferred_element_type=jnp.float32)
    probs = jax.nn.softmax(logits, axis=-1)
    col = lax.broadcasted_iota(jnp.int32, (T, E), 1)
    m1 = jnp.max(probs, axis=1, keepdims=True)
    i1 = jnp.min(jnp.where(probs == m1, col, E), axis=1, keepdims=True)
    masked = jnp.where(col == i1, -1.0, probs)
    m2 = jnp.max(masked, axis=1, keepdims=True)
    i2 = jnp.min(jnp.where(masked == m2, col, E), axis=1, keepdims=True)

    o1 = (col == i1).astype(jnp.float32)
    o2 = (col == i2).astype(jnp.float32)
    occ = o1 + o2
    incl = occ
    sh = 1
    while sh < T:
        shifted = jnp.concatenate(
            [jnp.zeros((sh, E), jnp.float32), incl[:-sh, :]], axis=0)
        incl = incl + shifted
        sh *= 2
    excl = incl - occ
    rank1 = jnp.sum(excl * o1, axis=1, keepdims=True).astype(jnp.int32)
    rank2 = jnp.sum(excl * o2, axis=1, keepdims=True).astype(jnp.int32)

    keep1 = rank1 < CAP
    keep2 = rank2 < CAP
    s1_ref[...] = jnp.where(keep1, i1 * CAP + rank1, TRASH)
    s2_ref[...] = jnp.where(keep2, i2 * CAP + rank2, TRASH)
    r1_ref[...] = i1 * CAP + jnp.minimum(rank1, CAP - 1)
    r2_ref[...] = i2 * CAP + jnp.minimum(rank2, CAP - 1)
    w1_ref[...] = jnp.broadcast_to(m1 * keep1.astype(jnp.float32), (T, 16))
    w2_ref[...] = jnp.broadcast_to(m2 * keep2.astype(jnp.float32), (T, 16))
    cnt_ref[...] = incl[T - 1:T, :].astype(jnp.int32)


def _router(hidden, w_router):
    return pl.pallas_call(
        _router_body,
        out_shape=(
            jax.ShapeDtypeStruct((T, 1), jnp.int32),
            jax.ShapeDtypeStruct((T, 1), jnp.int32),
            jax.ShapeDtypeStruct((T, 1), jnp.int32),
            jax.ShapeDtypeStruct((T, 1), jnp.int32),
            jax.ShapeDtypeStruct((T, 16), jnp.float32),
            jax.ShapeDtypeStruct((T, 16), jnp.float32),
            jax.ShapeDtypeStruct((1, E), jnp.int32),
        ),
    )(hidden, w_router)


@functools.lru_cache(maxsize=None)
def _sc_mesh():
    return plsc.VectorSubcoreMesh(
        core_axis_name="c", subcore_axis_name="s", num_cores=2,
        num_subcores=16)


@functools.lru_cache(maxsize=None)
def _dispatch_kernel():
    @functools.partial(
        pl.kernel,
        out_type=jax.ShapeDtypeStruct((DISP_ROWS, D_MODEL), jnp.float32),
        mesh=_sc_mesh(),
        scratch_types=[
            pltpu.VMEM((TPW,), jnp.int32),
            pltpu.VMEM((TPW,), jnp.int32),
            pltpu.VMEM((TPW, D_MODEL), jnp.float32),
            pltpu.SemaphoreType.DMA,
            pltpu.SemaphoreType.DMA,
        ],
    )
    def _dispatch(h_hbm, s1_hbm, s2_hbm, disp_hbm, idx1_v, idx2_v, rows_v,
                  sem1, sem2):
        wid = lax.axis_index("s") * 2 + lax.axis_index("c")
        base = wid * TPW
        pltpu.sync_copy(s1_hbm.at[pl.ds(base, TPW)], idx1_v)
        pltpu.sync_copy(s2_hbm.at[pl.ds(base, TPW)], idx2_v)
        pltpu.sync_copy(h_hbm.at[pl.ds(base, TPW)], rows_v)
        c1 = pltpu.async_copy(rows_v, disp_hbm.at[idx1_v], sem1)
        c2 = pltpu.async_copy(rows_v, disp_hbm.at[idx2_v], sem2)
        c1.wait()
        c2.wait()

    return _dispatch


def _gemm_body(cnt_ref, x_ref, wu_ref, wd_ref, y_ref):
    e = pl.program_id(0)
    cnt = cnt_ref[e]
    for ti in range(NC_BLK):
        start = ti * BC
        sl = pl.ds(start, BC)

        @pl.when(start >= cnt)
        def _():
            y_ref[sl, :] = jnp.zeros((BC, D_MODEL), jnp.float32)

        @pl.when(start < cnt)
        def _():
            rows = lax.broadcasted_iota(jnp.int32, (BC, 1), 0) + start
            x = jnp.where(rows < cnt, x_ref[sl, :], 0.0)
            acc = jnp.zeros((BC, D_MODEL), jnp.float32)
            for f in range(4):
                wu = wu_ref[0, :, f * 512:(f + 1) * 512]
                wd = wd_ref[0, f * 512:(f + 1) * 512, :]
                h = jax.nn.gelu(
                    jnp.dot(x, wu, preferred_element_type=jnp.float32))
                acc = acc + jnp.dot(h, wd, preferred_element_type=jnp.float32)
            y_ref[sl, :] = acc


def _gemm(counts, disp, w_up, w_down):
    return pl.pallas_call(
        _gemm_body,
        grid=(E,),
        in_specs=[
            pl.BlockSpec(memory_space=pltpu.SMEM),
            pl.BlockSpec((CAP, D_MODEL), lambda e: (e, 0)),
            pl.BlockSpec((1, D_MODEL, D_FF), lambda e: (e, 0, 0)),
            pl.BlockSpec((1, D_FF, D_MODEL), lambda e: (e, 0, 0)),
        ],
        out_specs=pl.BlockSpec((CAP, D_MODEL), lambda e: (e, 0)),
        out_shape=jax.ShapeDtypeStruct((NROWS, D_MODEL), jnp.float32),
    )(counts, disp, w_up, w_down)


@functools.lru_cache(maxsize=None)
def _combine_kernel():
    @functools.partial(
        pl.kernel,
        out_type=jax.ShapeDtypeStruct((T, D_MODEL), jnp.float32),
        mesh=_sc_mesh(),
        scratch_types=[
            pltpu.VMEM((CHUNK,), jnp.int32),
            pltpu.VMEM((CHUNK,), jnp.int32),
            pltpu.VMEM((TPW, 16), jnp.float32),
            pltpu.VMEM((TPW, 16), jnp.float32),
            pltpu.VMEM((CHUNK, D_MODEL), jnp.float32),
            pltpu.VMEM((CHUNK, D_MODEL), jnp.float32),
            pltpu.VMEM((CHUNK, D_MODEL), jnp.float32),
            pltpu.SemaphoreType.DMA,
            pltpu.SemaphoreType.DMA,
        ],
    )
    def _combine(ys_hbm, r1_hbm, r2_hbm, w1_hbm, w2_hbm, out_hbm,
                 idx1_v, idx2_v, w1_v, w2_v, bufa, bufb, outb, sema, semb):
        wid = lax.axis_index("s") * 2 + lax.axis_index("c")
        base = wid * TPW
        pltpu.sync_copy(w1_hbm.at[pl.ds(base, TPW)], w1_v)
        pltpu.sync_copy(w2_hbm.at[pl.ds(base, TPW)], w2_v)
        for ch in range(TPW // CHUNK):
            cb = base + ch * CHUNK
            pltpu.sync_copy(r1_hbm.at[pl.ds(cb, CHUNK)], idx1_v)
            pltpu.sync_copy(r2_hbm.at[pl.ds(cb, CHUNK)], idx2_v)
            ca = pltpu.async_copy(ys_hbm.at[idx1_v], bufa, sema)
            cbb = pltpu.async_copy(ys_hbm.at[idx2_v], bufb, semb)
            ca.wait()
            cbb.wait()

            def row(j, _):
                wa = w1_v[ch * CHUNK + j, :]
                wb = w2_v[ch * CHUNK + j, :]
                for cc in range(D_MODEL // 16):
                    sl = pl.ds(cc * 16, 16)
                    outb[j, sl] = bufa[j, sl] * wa + bufb[j, sl] * wb
                return 0

            lax.fori_loop(0, CHUNK, row, 0)
            pltpu.sync_copy(outb, out_hbm.at[pl.ds(cb, CHUNK)])

    return _combine


@jax.jit
def kernel(hidden_states, w_router, w_up, w_down):
    s1, s2, r1, r2, w1, w2, counts = _router(hidden_states, w_router)
    s1 = s1.reshape(T)
    s2 = s2.reshape(T)
    r1 = r1.reshape(T)
    r2 = r2.reshape(T)
    counts = counts.reshape(E)
    disp = _dispatch_kernel()(hidden_states, s1, s2)
    ys = _gemm(counts, disp, w_up, w_down)
    return _combine_kernel()(ys, r1, r2, w1, w2)


# speedup vs baseline: 2.2630x; 1.0030x over previous
import functools

import jax
import jax.numpy as jnp
from jax import lax
from jax.experimental import pallas as pl
from jax.experimental.pallas import tpu as pltpu
from jax.experimental.pallas import tpu_sc as plsc

E = 8
TOPK = 2
D_MODEL = 1024
D_FF = 2048
T = 2048
CAP = (T * TOPK // E) * 2
NROWS = E * CAP
TRASH = NROWS
DISP_ROWS = NROWS + 256

NW = 32
TPW = T // NW
CHUNK = 16
BC = 256
NC_BLK = CAP // BC


def _router_body(h_ref, wr_ref, s1_ref, s2_ref, r1_ref, r2_ref,
                 w1_ref, w2_ref, cnt_ref):
    logits = jnp.dot(h_ref[...], wr_ref[...], preferred_element_type=jnp.float32)
    probs = jax.nn.softmax(logits, axis=-1)
    col = lax.broadcasted_iota(jnp.int32, (T, E), 1)
    m1 = jnp.max(probs, axis=1, keepdims=True)
    i1 = jnp.min(jnp.where(probs == m1, col, E), axis=1, keepdims=True)
    masked = jnp.where(col == i1, -1.0, probs)
    m2 = jnp.max(masked, axis=1, keepdims=True)
    i2 = jnp.min(jnp.where(masked == m2, col, E), axis=1, keepdims=True)

    o1 = (col == i1).astype(jnp.float32)
    o2 = (col == i2).astype(jnp.float32)
    occ = o1 + o2
    incl = occ
    sh = 1
    while sh < T:
        shifted = jnp.concatenate(
            [jnp.zeros((sh, E), jnp.float32), incl[:-sh, :]], axis=0)
        incl = incl + shifted
        sh *= 2
    excl = incl - occ
    rank1 = jnp.sum(excl * o1, axis=1, keepdims=True).astype(jnp.int32)
    rank2 = jnp.sum(excl * o2, axis=1, keepdims=True).astype(jnp.int32)

    counts = incl[T - 1:T, :]
    cmin = jnp.min(counts, axis=1, keepdims=True)
    col8 = lax.broadcasted_iota(jnp.int32, (1, E), 1)
    ze = jnp.min(jnp.where(counts == cmin, col8, E), axis=1, keepdims=True)
    zr = ze * CAP + (CAP - 1)

    keep1 = rank1 < CAP
    keep2 = rank2 < CAP
    s1_ref[...] = jnp.where(keep1, i1 * CAP + rank1, TRASH)
    s2_ref[...] = jnp.where(keep2, i2 * CAP + rank2, TRASH)
    r1_ref[...] = jnp.where(keep1, i1 * CAP + rank1, zr)
    r2_ref[...] = jnp.where(keep2, i2 * CAP + rank2, zr)
    w1_ref[...] = jnp.broadcast_to(m1, (T, 128))
    w2_ref[...] = jnp.broadcast_to(m2, (T, 128))
    cnt_ref[...] = counts.astype(jnp.int32)


def _router(hidden, w_router):
    return pl.pallas_call(
        _router_body,
        out_shape=(
            jax.ShapeDtypeStruct((T, 1), jnp.int32),
            jax.ShapeDtypeStruct((T, 1), jnp.int32),
            jax.ShapeDtypeStruct((T, 1), jnp.int32),
            jax.ShapeDtypeStruct((T, 1), jnp.int32),
            jax.ShapeDtypeStruct((T, 128), jnp.float32),
            jax.ShapeDtypeStruct((T, 128), jnp.float32),
            jax.ShapeDtypeStruct((1, E), jnp.int32),
        ),
    )(hidden, w_router)


@functools.lru_cache(maxsize=None)
def _sc_mesh():
    return plsc.VectorSubcoreMesh(
        core_axis_name="c", subcore_axis_name="s", num_cores=2,
        num_subcores=16)


@functools.lru_cache(maxsize=None)
def _dispatch_kernel():
    @functools.partial(
        pl.kernel,
        out_type=(
            jax.ShapeDtypeStruct((DISP_ROWS, D_MODEL), jnp.float32),
            jax.ShapeDtypeStruct((NROWS + 16, 128), jnp.float32),
        ),
        mesh=_sc_mesh(),
        scratch_types=[
            pltpu.VMEM((TPW,), jnp.int32),
            pltpu.VMEM((TPW,), jnp.int32),
            pltpu.VMEM((TPW, D_MODEL), jnp.float32),
            pltpu.VMEM((TPW, 128), jnp.float32),
            pltpu.VMEM((TPW, 128), jnp.float32),
            pltpu.SemaphoreType.DMA,
            pltpu.SemaphoreType.DMA,
            pltpu.SemaphoreType.DMA,
            pltpu.SemaphoreType.DMA,
        ],
    )
    def _dispatch(h_hbm, s1_hbm, s2_hbm, w1_hbm, w2_hbm, disp_hbm, ws_hbm,
                  idx1_v, idx2_v, rows_v, w1_v, w2_v, sem1, sem2, sem3, sem4):
        wid = lax.axis_index("s") * 2 + lax.axis_index("c")
        base = wid * TPW
        pltpu.sync_copy(s1_hbm.at[pl.ds(base, TPW)], idx1_v)
        pltpu.sync_copy(s2_hbm.at[pl.ds(base, TPW)], idx2_v)
        pltpu.sync_copy(h_hbm.at[pl.ds(base, TPW)], rows_v)
        pltpu.sync_copy(w1_hbm.at[pl.ds(base, TPW)], w1_v)
        pltpu.sync_copy(w2_hbm.at[pl.ds(base, TPW)], w2_v)
        c1 = pltpu.async_copy(rows_v, disp_hbm.at[idx1_v], sem1)
        c2 = pltpu.async_copy(rows_v, disp_hbm.at[idx2_v], sem2)
        c3 = pltpu.async_copy(w1_v, ws_hbm.at[idx1_v], sem3)
        c4 = pltpu.async_copy(w2_v, ws_hbm.at[idx2_v], sem4)
        c1.wait()
        c2.wait()
        c3.wait()
        c4.wait()

    return _dispatch


def _gemm_body(cnt_ref, x_ref, wu_ref, wd_ref, ws_ref, y_ref):
    e = pl.program_id(0)
    cnt = cnt_ref[e]
    for ti in range(NC_BLK):
        start = ti * BC
        sl = pl.ds(start, BC)

        @pl.when(start >= cnt)
        def _():
            y_ref[sl, :] = jnp.zeros((BC, D_MODEL), jnp.float32)

        @pl.when(start < cnt)
        def _():
            rows = lax.broadcasted_iota(jnp.int32, (BC, 1), 0) + start
            x = x_ref[sl, :]
            acc = jnp.zeros((BC, D_MODEL), jnp.float32)
            for f in range(4):
                wu = wu_ref[0, :, f * 512:(f + 1) * 512]
                wd = wd_ref[0, f * 512:(f + 1) * 512, :]
                h = jax.nn.gelu(
                    jnp.dot(x, wu, preferred_element_type=jnp.float32))
                acc = acc + jnp.dot(h, wd, preferred_element_type=jnp.float32)
            w = ws_ref[sl, 0:1]
            y_ref[sl, :] = jnp.where(rows < cnt, acc * w, 0.0)


def _gemm(counts, disp, w_up, w_down, wslot):
    return pl.pallas_call(
        _gemm_body,
        grid=(E,),
        in_specs=[
            pl.BlockSpec(memory_space=pltpu.SMEM),
            pl.BlockSpec((CAP, D_MODEL), lambda e: (e, 0)),
            pl.BlockSpec((1, D_MODEL, D_FF), lambda e: (e, 0, 0)),
            pl.BlockSpec((1, D_FF, D_MODEL), lambda e: (e, 0, 0)),
            pl.BlockSpec((CAP, 128), lambda e: (e, 0)),
        ],
        out_specs=pl.BlockSpec((CAP, D_MODEL), lambda e: (e, 0)),
        out_shape=jax.ShapeDtypeStruct((NROWS, D_MODEL), jnp.float32),
    )(counts, disp, w_up, w_down, wslot)


@functools.lru_cache(maxsize=None)
def _combine_kernel():
    nch = TPW // CHUNK

    @functools.partial(
        pl.kernel,
        out_type=jax.ShapeDtypeStruct((T, D_MODEL), jnp.float32),
        mesh=_sc_mesh(),
        scratch_types=[
            pltpu.VMEM((TPW,), jnp.int32),
            pltpu.VMEM((TPW,), jnp.int32),
            pltpu.VMEM((2, CHUNK, D_MODEL), jnp.float32),
            pltpu.VMEM((2, CHUNK, D_MODEL), jnp.float32),
            pltpu.VMEM((CHUNK, D_MODEL), jnp.float32),
            pltpu.SemaphoreType.DMA,
            pltpu.SemaphoreType.DMA,
            pltpu.SemaphoreType.DMA,
            pltpu.SemaphoreType.DMA,
            pltpu.SemaphoreType.DMA,
        ],
    )
    def _combine(ys_hbm, r1_hbm, r2_hbm, out_hbm,
                 idx1_v, idx2_v, bufa, bufb, outb,
                 sa0, sa1, sb0, sb1, so):
        wid = lax.axis_index("s") * 2 + lax.axis_index("c")
        base = wid * TPW
        pltpu.sync_copy(r1_hbm.at[pl.ds(base, TPW)], idx1_v)
        pltpu.sync_copy(r2_hbm.at[pl.ds(base, TPW)], idx2_v)
        sas = (sa0, sa1)
        sbs = (sb0, sb1)

        def fire(ch):
            b = ch % 2
            ia = idx1_v.at[pl.ds(ch * CHUNK, CHUNK)]
            ib = idx2_v.at[pl.ds(ch * CHUNK, CHUNK)]
            ca = pltpu.async_copy(ys_hbm.at[ia], bufa.at[b], sas[b])
            cb = pltpu.async_copy(ys_hbm.at[ib], bufb.at[b], sbs[b])
            return ca, cb

        pend = fire(0)
        out_pend = None
        for ch in range(nch):
            b = ch % 2
            nxt = fire(ch + 1) if ch + 1 < nch else None
            pend[0].wait()
            pend[1].wait()
            if out_pend is not None:
                out_pend.wait()

            def row(j, _):
                for cc in range(D_MODEL // 16):
                    sl = pl.ds(cc * 16, 16)
                    outb[j, sl] = bufa[b, j, sl] + bufb[b, j, sl]
                return 0

            lax.fori_loop(0, CHUNK, row, 0)
            out_pend = pltpu.async_copy(
                outb, out_hbm.at[pl.ds(base + ch * CHUNK, CHUNK)], so)
            pend = nxt
        if out_pend is not None:
            out_pend.wait()

    return _combine


@jax.jit
def kernel(hidden_states, w_router, w_up, w_down):
    s1, s2, r1, r2, w1, w2, counts = _router(hidden_states, w_router)
    s1 = s1.reshape(T)
    s2 = s2.reshape(T)
    r1 = r1.reshape(T)
    r2 = r2.reshape(T)
    counts = counts.reshape(E)
    disp, wslot = _dispatch_kernel()(hidden_states, s1, s2, w1, w2)
    ys = _gemm(counts, disp, w_up, w_down, wslot)
    return _combine_kernel()(ys, r1, r2)
